# speedup vs baseline: 5.3947x; 5.3947x over previous
import functools

import jax
import jax.numpy as jnp
from jax import lax
from jax.experimental import pallas as pl
from jax.experimental.pallas import tpu as pltpu
from jax.experimental.pallas import tpu_sc as plsc

D = 1024
B = 16384
NC = 2
NS = 16
NW = NC * NS
CHUNK = 32
PER_W = B // NW
NCHUNK = PER_W // CHUNK
NBUF = 2


def _make_emb():
    mesh = plsc.VectorSubcoreMesh(core_axis_name="c", subcore_axis_name="s")

    @functools.partial(
        pl.kernel,
        mesh=mesh,
        out_type=jax.ShapeDtypeStruct((B, D), jnp.float32),
        scratch_types=[
            pltpu.VMEM((NCHUNK, CHUNK), jnp.int32),
            pltpu.VMEM((CHUNK, D), jnp.float32),
            pltpu.VMEM((CHUNK, D), jnp.float32),
            pltpu.SemaphoreType.DMA,
            pltpu.SemaphoreType.DMA,
            pltpu.SemaphoreType.DMA,
            pltpu.SemaphoreType.DMA,
        ],
    )
    def emb(ids_hbm, table_hbm, out_hbm,
            idx_v, rows0, rows1, gsem0, gsem1, psem0, psem1):
        wid = lax.axis_index("s") * NC + lax.axis_index("c")
        pltpu.sync_copy(ids_hbm.at[pl.ds(wid * NCHUNK, NCHUNK)], idx_v)

        rows = (rows0, rows1)
        gsem = (gsem0, gsem1)
        psem = (psem0, psem1)
        g = [None, None]
        p = [None, None]
        out_base = wid * PER_W

        g[0] = pltpu.async_copy(table_hbm.at[idx_v.at[0]], rows[0], gsem[0])
        for j in range(NCHUNK):
            b = j % NBUF
            nb = (j + 1) % NBUF
            if j + 1 < NCHUNK:
                if p[nb] is not None:
                    p[nb].wait()
                g[nb] = pltpu.async_copy(
                    table_hbm.at[idx_v.at[j + 1]], rows[nb], gsem[nb])
            g[b].wait()
            p[b] = pltpu.async_copy(
                rows[b], out_hbm.at[pl.ds(out_base + j * CHUNK, CHUNK)],
                psem[b])
        p[(NCHUNK - 1) % NBUF].wait()
        p[(NCHUNK - 2) % NBUF].wait()

    return emb


_emb = _make_emb()


@jax.jit
def kernel(input_ids, table):
    ids = input_ids.reshape(NW * NCHUNK, CHUNK).astype(jnp.int32)
    out = _emb(ids, table)
    return out.reshape(input_ids.shape + (D,))


# speedup vs baseline: 5.4457x; 1.0095x over previous
import functools

import jax
import jax.numpy as jnp
from jax import lax
from jax.experimental import pallas as pl
from jax.experimental.pallas import tpu as pltpu
from jax.experimental.pallas import tpu_sc as plsc

D = 1024
B = 16384
NC = 2
NS = 16
NW = NC * NS
CHUNK = 32
PER_W = B // NW
NCHUNK = PER_W // CHUNK
NBUF = 3


def _make_emb():
    mesh = plsc.VectorSubcoreMesh(core_axis_name="c", subcore_axis_name="s")

    @functools.partial(
        pl.kernel,
        mesh=mesh,
        out_type=jax.ShapeDtypeStruct((B, D), jnp.float32),
        scratch_types=(
            [pltpu.VMEM((NCHUNK, CHUNK), jnp.int32)]
            + [pltpu.VMEM((CHUNK, D), jnp.float32)] * NBUF
            + [pltpu.SemaphoreType.DMA] * (2 * NBUF)
        ),
    )
    def emb(ids_hbm, table_hbm, out_hbm, idx_v, *bufs):
        rows = bufs[:NBUF]
        gsem = bufs[NBUF:2 * NBUF]
        psem = bufs[2 * NBUF:]
        wid = lax.axis_index("s") * NC + lax.axis_index("c")
        pltpu.sync_copy(ids_hbm.at[pl.ds(wid * NCHUNK, NCHUNK)], idx_v)

        g = [None] * NBUF
        p = [None] * NBUF
        out_base = wid * PER_W

        for j in range(NBUF - 1):
            g[j] = pltpu.async_copy(table_hbm.at[idx_v.at[j]], rows[j],
                                    gsem[j])
        for j in range(NCHUNK):
            b = j % NBUF
            nb = (j + NBUF - 1) % NBUF
            jn = j + NBUF - 1
            if jn < NCHUNK:
                if p[nb] is not None:
                    p[nb].wait()
                g[nb] = pltpu.async_copy(
                    table_hbm.at[idx_v.at[jn]], rows[nb], gsem[nb])
            g[b].wait()
            p[b] = pltpu.async_copy(
                rows[b], out_hbm.at[pl.ds(out_base + j * CHUNK, CHUNK)],
                psem[b])
        for j in range(max(0, NCHUNK - NBUF), NCHUNK):
            p[j % NBUF].wait()

    return emb


_emb = _make_emb()


@jax.jit
def kernel(input_ids, table):
    ids = input_ids.reshape(NW * NCHUNK, CHUNK).astype(jnp.int32)
    out = _emb(ids, table)
    return out.reshape(input_ids.shape + (D,))


# speedup vs baseline: 5.4886x; 1.0079x over previous
import functools

import jax
import jax.numpy as jnp
from jax import lax
from jax.experimental import pallas as pl
from jax.experimental.pallas import tpu as pltpu
from jax.experimental.pallas import tpu_sc as plsc

D = 1024
B = 16384
NC = 2
NS = 16
NW = NC * NS
CHUNK = 16
PER_W = B // NW
NCHUNK = PER_W // CHUNK
NBUF = 6


def _make_emb():
    mesh = plsc.VectorSubcoreMesh(core_axis_name="c", subcore_axis_name="s")

    @functools.partial(
        pl.kernel,
        mesh=mesh,
        out_type=jax.ShapeDtypeStruct((B, D), jnp.float32),
        scratch_types=(
            [pltpu.VMEM((NCHUNK, CHUNK), jnp.int32)]
            + [pltpu.VMEM((CHUNK, D), jnp.float32)] * NBUF
            + [pltpu.SemaphoreType.DMA] * (2 * NBUF)
        ),
    )
    def emb(ids_hbm, table_hbm, out_hbm, idx_v, *bufs):
        sid = lax.axis_index("s")
        rows = bufs[:NBUF]
        gsem = bufs[NBUF:2 * NBUF]
        psem = bufs[2 * NBUF:]
        wid = sid * NC + lax.axis_index("c")
        pltpu.sync_copy(ids_hbm.at[pl.ds(wid * NCHUNK, NCHUNK)], idx_v)

        g = [None] * NBUF
        p = [None] * NBUF
        out_base = wid * PER_W

        for j in range(NBUF - 1):
            g[j] = pltpu.async_copy(table_hbm.at[idx_v.at[j]], rows[j],
                                    gsem[j])
        for j in range(NCHUNK):
            b = j % NBUF
            nb = (j + NBUF - 1) % NBUF
            jn = j + NBUF - 1
            if jn < NCHUNK:
                if p[nb] is not None:
                    p[nb].wait()
                g[nb] = pltpu.async_copy(
                    table_hbm.at[idx_v.at[jn]], rows[nb], gsem[nb])
            g[b].wait()
            p[b] = pltpu.async_copy(
                rows[b], out_hbm.at[pl.ds(out_base + j * CHUNK, CHUNK)],
                psem[b])
        for j in range(max(0, NCHUNK - NBUF), NCHUNK):
            p[j % NBUF].wait()

    return emb


_emb = _make_emb()


@jax.jit
def kernel(input_ids, table):
    ids = input_ids.reshape(NW * NCHUNK, CHUNK).astype(jnp.int32)
    out = _emb(ids, table)
    return out.reshape(input_ids.shape + (D,))
